# Initial kernel scaffold; baseline (speedup 1.0000x reference)
import jax
import jax.numpy as jnp
from jax.experimental import pallas as pl


def kernel(x, w_ih, w_hh, b_ih, b_hh):
    raise NotImplementedError("write your pallas kernel here")


# speedup vs baseline: 12.5391x; 12.5391x over previous
import math

import jax
import jax.numpy as jnp
from jax.experimental import pallas as pl

FEAT = 512
WIN = 12
NOFF = 23
RAD = 11
TILE = 256
HALO = TILE + 24


def _dot(a, b):
    return jax.lax.dot_general(
        a, b, (((1,), (1,)), ((), ())), preferred_element_type=jnp.float32
    )


def _gru_kernel(x_ref, wih_ref, whh_ref, bih_ref, bhh_ref, o_ref, *, t_total):
    j = pl.program_id(1)
    base = j * TILE
    D = FEAT

    halo = x_ref[0, pl.ds(base, HALO), :]
    center = halo[RAD:RAD + TILE, :]

    S = _dot(center, halo) / math.sqrt(D)
    row = jax.lax.broadcasted_iota(jnp.int32, (TILE, HALO), 0)
    col = jax.lax.broadcasted_iota(jnp.int32, (TILE, HALO), 1)
    cols = []
    for o in range(NOFF):
        m = col == row + o
        cols.append(jnp.sum(jnp.where(m, S, 0.0), axis=1, keepdims=True))
    Sb = jnp.concatenate(cols, axis=1)

    r23 = jax.lax.broadcasted_iota(jnp.int32, (TILE, NOFF), 0)
    o23 = jax.lax.broadcasted_iota(jnp.int32, (TILE, NOFF), 1)
    nbr = base + r23 + o23 - RAD
    valid = (nbr >= 0) & (nbr < t_total)
    Sb = jnp.where(valid, Sb, -1e9)

    sel = jnp.zeros((TILE, NOFF), jnp.bool_)
    Sw = Sb
    for _ in range(WIN):
        m = jnp.max(Sw, axis=1, keepdims=True)
        eq = Sw == m
        first = jnp.min(jnp.where(eq, o23, NOFF), axis=1, keepdims=True)
        oh = o23 == first
        sel = sel | oh
        Sw = jnp.where(oh, -jnp.inf, Sw)
    self_f = sel.astype(jnp.float32)

    a23 = jax.lax.broadcasted_iota(jnp.int32, (NOFF, NOFF), 0)
    b23 = jax.lax.broadcasted_iota(jnp.int32, (NOFF, NOFF), 1)
    ltri = (a23 < b23).astype(jnp.float32)
    ordv = jax.lax.dot_general(
        self_f, ltri, (((1,), (0,)), ((), ())),
        preferred_element_type=jnp.float32)

    wih = wih_ref[...]
    whh = whh_ref[...]
    bih = bih_ref[...]
    bhh = bhh_ref[...]
    G = _dot(halo, wih)

    h = jnp.zeros((TILE, D), jnp.float32)
    off_f = o23.astype(jnp.float32)
    for w in range(WIN):
        ohw = jnp.where(sel & (ordv == float(w)), 1.0, 0.0)
        off = jnp.sum(ohw * off_f, axis=1, keepdims=True).astype(jnp.int32)
        P = (col == row + off).astype(jnp.float32)
        gi = jax.lax.dot_general(
            P, G, (((1,), (0,)), ((), ())),
            preferred_element_type=jnp.float32) + bih
        gh = _dot(h, whh) + bhh
        r = jax.nn.sigmoid(gi[:, :D] + gh[:, :D])
        z = jax.nn.sigmoid(gi[:, D:2 * D] + gh[:, D:2 * D])
        n = jnp.tanh(gi[:, 2 * D:] + r * gh[:, 2 * D:])
        h = (1.0 - z) * n + z * h

    o_ref[0, :, :] = h + center


def kernel(x, w_ih, w_hh, b_ih, b_hh):
    B, T, D = x.shape
    nt = T // TILE
    pad_r = (HALO - TILE) - RAD
    x_pad = jnp.pad(x, ((0, 0), (RAD, pad_r), (0, 0)))
    import functools
    kern = functools.partial(_gru_kernel, t_total=T)
    out = pl.pallas_call(
        kern,
        grid=(B, nt),
        in_specs=[
            pl.BlockSpec((1, T + (HALO - TILE), D), lambda b, j: (b, 0, 0)),
            pl.BlockSpec((3 * D, D), lambda b, j: (0, 0)),
            pl.BlockSpec((3 * D, D), lambda b, j: (0, 0)),
            pl.BlockSpec((1, 3 * D), lambda b, j: (0, 0)),
            pl.BlockSpec((1, 3 * D), lambda b, j: (0, 0)),
        ],
        out_specs=pl.BlockSpec((1, TILE, D), lambda b, j: (b, j, 0)),
        out_shape=jax.ShapeDtypeStruct((B, T, D), x.dtype),
    )(x_pad, w_ih, w_hh, b_ih.reshape(1, -1), b_hh.reshape(1, -1))
    return out


# speedup vs baseline: 13.1100x; 1.0455x over previous
import math

import jax
import jax.numpy as jnp
from jax.experimental import pallas as pl

FEAT = 512
WIN = 12
NOFF = 23
RAD = 11
TILE = 256
HALO = TILE + 24


def _dot(a, b):
    return jax.lax.dot_general(
        a, b, (((1,), (1,)), ((), ())), preferred_element_type=jnp.float32
    )


def _gru_kernel(x_ref, wih_ref, whh_ref, bih_ref, bhh_ref, o_ref, *, t_total):
    j = pl.program_id(1)
    base = j * TILE
    D = FEAT

    halo = x_ref[0, pl.ds(base, HALO), :]
    center = halo[RAD:RAD + TILE, :]

    S = _dot(center, halo) / math.sqrt(D)
    row = jax.lax.broadcasted_iota(jnp.int32, (TILE, HALO), 0)
    col = jax.lax.broadcasted_iota(jnp.int32, (TILE, HALO), 1)
    cols = []
    for o in range(NOFF):
        m = col == row + o
        cols.append(jnp.sum(jnp.where(m, S, 0.0), axis=1, keepdims=True))
    Sb = jnp.concatenate(cols, axis=1)

    r23 = jax.lax.broadcasted_iota(jnp.int32, (TILE, NOFF), 0)
    o23 = jax.lax.broadcasted_iota(jnp.int32, (TILE, NOFF), 1)
    nbr = base + r23 + o23 - RAD
    valid = (nbr >= 0) & (nbr < t_total)
    Sb = jnp.where(valid, Sb, -1e9)

    sel = jnp.zeros((TILE, NOFF), jnp.bool_)
    Sw = Sb
    for _ in range(WIN):
        m = jnp.max(Sw, axis=1, keepdims=True)
        eq = Sw == m
        first = jnp.min(jnp.where(eq, o23, NOFF), axis=1, keepdims=True)
        oh = o23 == first
        sel = sel | oh
        Sw = jnp.where(oh, -jnp.inf, Sw)
    self_f = sel.astype(jnp.float32)

    a23 = jax.lax.broadcasted_iota(jnp.int32, (NOFF, NOFF), 0)
    b23 = jax.lax.broadcasted_iota(jnp.int32, (NOFF, NOFF), 1)
    ltri = (a23 < b23).astype(jnp.float32)
    ordv = jax.lax.dot_general(
        self_f, ltri, (((1,), (0,)), ((), ())),
        preferred_element_type=jnp.float32)

    wih = wih_ref[...]
    whh = whh_ref[...].astype(jnp.bfloat16)
    bih = bih_ref[...]
    bhh = bhh_ref[...]
    G = _dot(halo, wih).astype(jnp.bfloat16)

    h = jnp.zeros((TILE, D), jnp.float32)
    off_f = o23.astype(jnp.float32)
    for w in range(WIN):
        ohw = jnp.where(sel & (ordv == float(w)), 1.0, 0.0)
        off = jnp.sum(ohw * off_f, axis=1, keepdims=True).astype(jnp.int32)
        P = (col == row + off).astype(jnp.bfloat16)
        gi = jax.lax.dot_general(
            P, G, (((1,), (0,)), ((), ())),
            preferred_element_type=jnp.float32) + bih
        gh = _dot(h.astype(jnp.bfloat16), whh) + bhh
        r = jax.nn.sigmoid(gi[:, :D] + gh[:, :D])
        z = jax.nn.sigmoid(gi[:, D:2 * D] + gh[:, D:2 * D])
        n = jnp.tanh(gi[:, 2 * D:] + r * gh[:, 2 * D:])
        h = (1.0 - z) * n + z * h

    o_ref[0, :, :] = h + center


def kernel(x, w_ih, w_hh, b_ih, b_hh):
    B, T, D = x.shape
    nt = T // TILE
    pad_r = (HALO - TILE) - RAD
    x_pad = jnp.pad(x, ((0, 0), (RAD, pad_r), (0, 0)))
    import functools
    kern = functools.partial(_gru_kernel, t_total=T)
    out = pl.pallas_call(
        kern,
        grid=(B, nt),
        in_specs=[
            pl.BlockSpec((1, T + (HALO - TILE), D), lambda b, j: (b, 0, 0)),
            pl.BlockSpec((3 * D, D), lambda b, j: (0, 0)),
            pl.BlockSpec((3 * D, D), lambda b, j: (0, 0)),
            pl.BlockSpec((1, 3 * D), lambda b, j: (0, 0)),
            pl.BlockSpec((1, 3 * D), lambda b, j: (0, 0)),
        ],
        out_specs=pl.BlockSpec((1, TILE, D), lambda b, j: (b, j, 0)),
        out_shape=jax.ShapeDtypeStruct((B, T, D), x.dtype),
    )(x_pad, w_ih, w_hh, b_ih.reshape(1, -1), b_hh.reshape(1, -1))
    return out
